# Initial kernel scaffold; baseline (speedup 1.0000x reference)
import jax
import jax.numpy as jnp
from jax.experimental import pallas as pl


def kernel(x, W_cont, b_cont, tables, W_out, b_out):
    raise NotImplementedError("write your pallas kernel here")


# speedup vs baseline: 6.6187x; 6.6187x over previous
import functools

import jax
import jax.numpy as jnp
from jax import lax
from jax.experimental import pallas as pl
from jax.experimental.pallas import tpu as pltpu
from jax.experimental.pallas import tpu_sc as plsc

B = 16384
N_CONT = 13
N_CAT = 26
V1 = 1001
D = 128

NC = 2
NS = 16
NW = NC * NS
BPW = B // NW
S = 2
ROWS = S * N_CAT
CH = BPW // S


def _sc_body(tab_hbm, fidx_hbm, out_hbm, idx_v, ring_v, out_v, sem0, sem1):
    wid = lax.axis_index("s") * NC + lax.axis_index("c")
    pltpu.sync_copy(fidx_hbm.at[wid], idx_v)
    sems = (sem0, sem1)

    def start(c, buf):
        pltpu.async_copy(tab_hbm.at[idx_v.at[c]], ring_v.at[buf], sems[buf])

    def wait(buf):
        pltpu.make_async_copy(
            tab_hbm.at[idx_v.at[0]], ring_v.at[buf], sems[buf]
        ).wait()

    def accum(c, buf):
        for s in range(S):
            acc = [ring_v[buf, s * N_CAT, pl.ds(v * 16, 16)] for v in range(8)]
            for i in range(1, N_CAT):
                for v in range(8):
                    acc[v] = acc[v] + ring_v[buf, s * N_CAT + i, pl.ds(v * 16, 16)]
            row = pl.multiple_of((c * S + s) * D, 128)
            for v in range(8):
                out_v[pl.ds(row + v * 16, 16)] = acc[v]

    start(0, 0)
    start(1, 1)

    def pair(p, carry):
        c0 = 2 * p
        wait(0)
        accum(c0, 0)

        @pl.when(p + 1 < CH // 2)
        def _():
            start(c0 + 2, 0)

        wait(1)
        accum(c0 + 1, 1)

        @pl.when(p + 1 < CH // 2)
        def _():
            start(c0 + 3, 1)

        return carry

    lax.fori_loop(0, CH // 2, pair, None)
    pltpu.sync_copy(out_v, out_hbm.at[wid])


_sc_embed = functools.partial(
    pl.kernel,
    out_type=jax.ShapeDtypeStruct((NW, BPW * D), jnp.float32),
    mesh=plsc.VectorSubcoreMesh(core_axis_name="c", subcore_axis_name="s"),
    scratch_types=[
        pltpu.VMEM((CH, ROWS), jnp.int32),
        pltpu.VMEM((2, ROWS, D), jnp.float32),
        pltpu.VMEM((BPW * D,), jnp.float32),
        pltpu.SemaphoreType.DMA,
        pltpu.SemaphoreType.DMA,
    ],
)(_sc_body)


BLK = 2048


def _dense_body(cont_ref, cat_ref, wc_ref, bc_ref, wo_ref, bo_ref, out_ref):
    c1 = jnp.dot(cont_ref[...], wc_ref[...], preferred_element_type=jnp.float32)
    c1 = c1 + bc_ref[...]
    wo = wo_ref[...]
    h = jnp.dot(c1, wo[:D], preferred_element_type=jnp.float32)
    h = h + jnp.dot(cat_ref[...], wo[D:], preferred_element_type=jnp.float32)
    out_ref[...] = h + bo_ref[...]


def _dense(cont, cat_sum, W_cont, b_cont, W_out, b_out):
    return pl.pallas_call(
        _dense_body,
        grid=(B // BLK,),
        in_specs=[
            pl.BlockSpec((BLK, N_CONT), lambda i: (i, 0)),
            pl.BlockSpec((BLK, D), lambda i: (i, 0)),
            pl.BlockSpec((N_CONT, D), lambda i: (0, 0)),
            pl.BlockSpec((1, D), lambda i: (0, 0)),
            pl.BlockSpec((2 * D, D), lambda i: (0, 0)),
            pl.BlockSpec((1, D), lambda i: (0, 0)),
        ],
        out_specs=pl.BlockSpec((BLK, D), lambda i: (i, 0)),
        out_shape=jax.ShapeDtypeStruct((B, D), jnp.float32),
    )(cont, cat_sum, W_cont, b_cont, W_out, b_out)


def kernel(x, W_cont, b_cont, tables, W_out, b_out):
    x = x.astype(jnp.int32)
    cont = x[:, :N_CONT].astype(jnp.float32)
    offs = 1 + V1 * jnp.arange(N_CAT, dtype=jnp.int32)
    fidx = (x[:, N_CONT:] + offs).reshape(NW, CH, ROWS)
    tab_flat = tables.reshape(N_CAT * V1, D)
    cat_sum = _sc_embed(tab_flat, fidx).reshape(B, D)
    return _dense(
        cont, cat_sum, W_cont, b_cont.reshape(1, D), W_out, b_out.reshape(1, D)
    )
